# baseline (speedup 1.0000x reference)
import functools

import jax
import jax.numpy as jnp
from jax import lax
from jax.experimental import pallas as pl
from jax.experimental.pallas import tpu as pltpu
from jax.experimental.pallas import tpu_sc as plsc

POOL_SIZE = 30
PROMPT_LEN = 5
TOP_K = 5
EMBED_DIM = 768
BATCH = 128
TOKENS = 196
BATCH_BLK = 16

_NC = 2
_NS = 16
_NW = _NC * _NS
_ROWS = BATCH * TOP_K
_ROW_W = PROMPT_LEN * EMBED_DIM
_CHUNK = 16
_NCHUNK = _ROWS // _CHUNK


def _topk_body(x_hbm, p_hbm, ids_ref, xb, pb, sem_x, sem_p):
    i = pl.program_id(0)
    nb = pl.num_programs(0)

    @pl.when(i == 0)
    def _():
        pltpu.make_async_copy(p_hbm, pb, sem_p).start()
        pltpu.make_async_copy(
            x_hbm.at[pl.ds(0, BATCH_BLK)], xb.at[0], sem_x.at[0]).start()

    @pl.when(i + 1 < nb)
    def _():
        pltpu.make_async_copy(
            x_hbm.at[pl.ds((i + 1) * BATCH_BLK, BATCH_BLK)],
            xb.at[(i + 1) % 2], sem_x.at[(i + 1) % 2]).start()

    pltpu.make_async_copy(
        x_hbm.at[pl.ds(i * BATCH_BLK, BATCH_BLK)],
        xb.at[i % 2], sem_x.at[i % 2]).wait()

    @pl.when(i == 0)
    def _():
        pltpu.make_async_copy(p_hbm, pb, sem_p).wait()

    x = xb[i % 2]
    embed_keys = jnp.max(x, axis=1) + 2.0 * (jnp.sum(x, axis=1) * (1.0 / TOKENS))

    keys = jnp.sum(pb[...], axis=1) * (1.0 / PROMPT_LEN)

    def l2(v):
        ss = jnp.sum(v * v, axis=1, keepdims=True)
        return v * lax.rsqrt(jnp.maximum(ss, 1e-12))

    sim = jnp.dot(l2(embed_keys), l2(keys).T,
                  preferred_element_type=jnp.float32)

    col = lax.broadcasted_iota(jnp.int32, sim.shape, 1)
    picked = []
    for _ in range(TOP_K):
        m = jnp.max(sim, axis=1, keepdims=True)
        idx = jnp.min(jnp.where(sim == m, col, POOL_SIZE), axis=1)
        picked.append(idx)
        sim = jnp.where(col == idx[:, None], -jnp.inf, sim)
    ids_ref[...] = jnp.stack(picked, axis=1)


def _topk_ids(input_embed, prompt):
    return pl.pallas_call(
        _topk_body,
        grid=(BATCH // BATCH_BLK,),
        in_specs=[
            pl.BlockSpec(memory_space=pl.ANY),
            pl.BlockSpec(memory_space=pl.ANY),
        ],
        out_specs=pl.BlockSpec((BATCH_BLK, TOP_K), lambda i: (i, 0)),
        out_shape=jax.ShapeDtypeStruct((BATCH, TOP_K), jnp.int32),
        scratch_shapes=[
            pltpu.VMEM((2, BATCH_BLK, TOKENS, EMBED_DIM), jnp.float32),
            pltpu.VMEM((POOL_SIZE, PROMPT_LEN, EMBED_DIM), jnp.float32),
            pltpu.SemaphoreType.DMA((2,)),
            pltpu.SemaphoreType.DMA,
        ],
    )(input_embed, prompt)


def _sc_gather(table, idx2d):
    mesh = plsc.VectorSubcoreMesh(core_axis_name="c", subcore_axis_name="s")

    @functools.partial(
        pl.kernel,
        mesh=mesh,
        out_type=jax.ShapeDtypeStruct((_ROWS, _ROW_W), jnp.float32),
        scratch_types=[
            pltpu.VMEM((_CHUNK,), jnp.int32),
            pltpu.VMEM((_CHUNK, _ROW_W), jnp.float32),
            pltpu.SemaphoreType.DMA,
        ],
    )
    def k(table_hbm, idx_hbm, out_hbm, idx_v, rows_v, sem):
        wid = lax.axis_index("s") * _NC + lax.axis_index("c")

        def do_chunk(t):
            pltpu.sync_copy(idx_hbm.at[t], idx_v)
            pltpu.async_copy(table_hbm.at[idx_v], rows_v, sem).wait()
            pltpu.sync_copy(rows_v, out_hbm.at[pl.ds(t * _CHUNK, _CHUNK)])

        do_chunk(wid)

        @pl.when(wid < _NCHUNK - _NW)
        def _():
            do_chunk(wid + _NW)

    return k(table, idx2d)


def kernel(input_embed, prompt):
    ids = _topk_ids(input_embed, prompt)
    idx2d = ids.reshape(_NCHUNK, _CHUNK)
    table = prompt.reshape(POOL_SIZE, _ROW_W)
    rows = _sc_gather(table, idx2d)
    return rows.reshape(BATCH, TOP_K * PROMPT_LEN, EMBED_DIM)


# speedup vs baseline: 2.4880x; 2.4880x over previous
import functools

import jax
import jax.numpy as jnp
from jax import lax
from jax.experimental import pallas as pl
from jax.experimental.pallas import tpu as pltpu
from jax.experimental.pallas import tpu_sc as plsc

POOL_SIZE = 30
PROMPT_LEN = 5
TOP_K = 5
EMBED_DIM = 768
BATCH = 128
TOKENS = 196
TOK_BLK = 28

_NSLAB = TOP_K * PROMPT_LEN
_NSLAB_PAD = 32


def _topk_body(x_ref, p_ref, idx_ref, maxs, sums):
    i = pl.program_id(0)
    n = pl.num_programs(0)
    x = x_ref[...]
    bmax = jnp.max(x, axis=0)
    bsum = jnp.sum(x, axis=0)

    @pl.when(i == 0)
    def _():
        maxs[...] = bmax
        sums[...] = bsum

    @pl.when(i > 0)
    def _():
        maxs[...] = jnp.maximum(maxs[...], bmax)
        sums[...] = sums[...] + bsum

    @pl.when(i == n - 1)
    def _():
        embed_keys = maxs[...] + 2.0 * (sums[...] * (1.0 / TOKENS))
        keys = jnp.sum(p_ref[...], axis=0) * (1.0 / PROMPT_LEN)

        def l2(v):
            ss = jnp.sum(v * v, axis=1, keepdims=True)
            return v * lax.rsqrt(jnp.maximum(ss, 1e-12))

        sim = lax.dot_general(
            l2(embed_keys), l2(keys),
            dimension_numbers=(((1,), (1,)), ((), ())),
            preferred_element_type=jnp.float32)

        col = lax.broadcasted_iota(jnp.int32, sim.shape, 1)
        rows = []
        for _ in range(TOP_K):
            m = jnp.max(sim, axis=1, keepdims=True)
            idx = jnp.min(jnp.where(sim == m, col, POOL_SIZE), axis=1)
            rows.extend((idx + t * POOL_SIZE)[None, :] for t in range(PROMPT_LEN))
            sim = jnp.where(col == idx[:, None], -jnp.inf, sim)
        rows.append(jnp.zeros((_NSLAB_PAD - _NSLAB, BATCH), jnp.int32))
        idx_ref[...] = jnp.concatenate(rows, axis=0)


def _topk_idx(x_t, p_t):
    return pl.pallas_call(
        _topk_body,
        grid=(TOKENS // TOK_BLK,),
        in_specs=[
            pl.BlockSpec((TOK_BLK, BATCH, EMBED_DIM), lambda i: (i, 0, 0)),
            pl.BlockSpec((PROMPT_LEN, POOL_SIZE, EMBED_DIM), lambda i: (0, 0, 0)),
        ],
        out_specs=pl.BlockSpec((_NSLAB_PAD, BATCH), lambda i: (0, 0)),
        out_shape=jax.ShapeDtypeStruct((_NSLAB_PAD, BATCH), jnp.int32),
        scratch_shapes=[
            pltpu.VMEM((BATCH, EMBED_DIM), jnp.float32),
            pltpu.VMEM((BATCH, EMBED_DIM), jnp.float32),
        ],
    )(x_t, p_t)


def _sc_gather(table, idx):
    mesh = plsc.VectorSubcoreMesh(core_axis_name="c", subcore_axis_name="s")

    @functools.partial(
        pl.kernel,
        mesh=mesh,
        out_type=jax.ShapeDtypeStruct((_NSLAB, BATCH, EMBED_DIM), jnp.float32),
        scratch_types=[
            pltpu.VMEM((BATCH,), jnp.int32),
            pltpu.VMEM((BATCH, EMBED_DIM), jnp.float32),
            pltpu.SemaphoreType.DMA,
        ],
    )
    def k(table_hbm, idx_hbm, out_hbm, idx_v, rows_v, sem):
        wid = lax.axis_index("s") * 2 + lax.axis_index("c")

        @pl.when(wid < _NSLAB)
        def _():
            pltpu.sync_copy(idx_hbm.at[wid], idx_v)
            pltpu.async_copy(table_hbm.at[idx_v], rows_v, sem).wait()
            pltpu.sync_copy(rows_v, out_hbm.at[wid])

    return k(table, idx)


def kernel(input_embed, prompt):
    x_t = jnp.transpose(input_embed, (1, 0, 2))
    p_t = jnp.transpose(prompt, (1, 0, 2))
    idx = _topk_idx(x_t, p_t)
    table = p_t.reshape(PROMPT_LEN * POOL_SIZE, EMBED_DIM)
    out_t = _sc_gather(table, idx)
    return jnp.transpose(out_t, (1, 0, 2))
